# baseline (device time: 38557 ns/iter reference)
import jax
import jax.numpy as jnp
from jax import lax
from jax.experimental import pallas as pl
from jax.experimental.pallas import tpu as pltpu


def kernel(x, dest):
    m, n = x.shape
    d2 = dest.reshape(8, 128)

    def body(x_ref, d_ref, xg_ref, dg_ref, sem_sx, sem_rx, sem_sd, sem_rd):
        my_x = lax.axis_index("x")
        my_y = lax.axis_index("y")
        my_z = lax.axis_index("z")
        peer = (1 - my_x, my_y, my_z)

        barrier = pltpu.get_barrier_semaphore()
        pl.semaphore_signal(
            barrier, inc=1, device_id=peer, device_id_type=pl.DeviceIdType.MESH
        )
        pl.semaphore_wait(barrier, 1)

        def exchange(slot):
            rx = pltpu.make_async_remote_copy(
                src_ref=x_ref,
                dst_ref=xg_ref.at[slot],
                send_sem=sem_sx,
                recv_sem=sem_rx,
                device_id=peer,
                device_id_type=pl.DeviceIdType.MESH,
            )
            rd = pltpu.make_async_remote_copy(
                src_ref=d_ref,
                dst_ref=dg_ref.at[slot],
                send_sem=sem_sd,
                recv_sem=sem_rd,
                device_id=peer,
                device_id_type=pl.DeviceIdType.MESH,
            )
            rx.start()
            rd.start()
            xg_ref[slot] = x_ref[...]
            dg_ref[slot] = d_ref[...]
            rx.wait()
            rd.wait()

        @pl.when(my_x == 0)
        def _():
            exchange(0)

        @pl.when(my_x == 1)
        def _():
            exchange(1)

    xg, dg = pl.pallas_call(
        body,
        out_shape=(
            jax.ShapeDtypeStruct((2, m, n), jnp.float32),
            jax.ShapeDtypeStruct((2, 8, 128), jnp.int32),
        ),
        in_specs=[
            pl.BlockSpec(memory_space=pltpu.VMEM),
            pl.BlockSpec(memory_space=pltpu.VMEM),
        ],
        out_specs=(
            pl.BlockSpec(memory_space=pltpu.VMEM),
            pl.BlockSpec(memory_space=pltpu.VMEM),
        ),
        scratch_shapes=[
            pltpu.SemaphoreType.DMA,
            pltpu.SemaphoreType.DMA,
            pltpu.SemaphoreType.DMA,
            pltpu.SemaphoreType.DMA,
        ],
        compiler_params=pltpu.CompilerParams(collective_id=0),
    )(x, d2)

    my_x = lax.axis_index("x")
    x_glob = xg.reshape(2 * m, n)
    dest_glob = dg.reshape(2 * m)
    order = jnp.argsort(dest_glob, stable=True)
    my_idx = lax.dynamic_slice(order, (my_x * m,), (m,))
    return jnp.take(x_glob, my_idx, axis=0)
